# baseline (speedup 1.0000x reference)
import functools

import jax
import jax.numpy as jnp
from jax import lax
from jax.experimental import pallas as pl
from jax.experimental.pallas import tpu as pltpu
from jax.experimental.pallas import tpu_sc as plsc

N = 100000
E = 6400000
NC, NS, L = 2, 16, 16
NW = NC * NS
EW = E // NW
K = 800
K3 = 3 * K
K6 = 6 * K
NB = EW // K
G = K // L
ROWS_T = 6256
ROWS_LAST = N - (NS - 1) * ROWS_T

_MESH = plsc.VectorSubcoreMesh(
    core_axis_name="c", subcore_axis_name="s", num_cores=NC, num_subcores=NS)

_f32 = jnp.float32
_i32 = jnp.int32


def _rsqrt(d2):
    i = plsc.bitcast(d2, _i32)
    i = jnp.int32(0x5F3759DF) - lax.shift_right_logical(i, 1)
    r = plsc.bitcast(i, _f32)
    hd = 0.5 * d2
    for _ in range(3):
        r = r * (1.5 - hd * r * r)
    return r


@functools.partial(
    pl.kernel,
    out_type=(
        jax.ShapeDtypeStruct((NC, N * 3), _f32),
        jax.ShapeDtypeStruct((E * 3,), _f32),
        jax.ShapeDtypeStruct((E * 3,), _f32),
        jax.ShapeDtypeStruct((E * 6,), _f32),
        jax.ShapeDtypeStruct((E * 6,), _f32),
        jax.ShapeDtypeStruct((E,), _f32),
        jax.ShapeDtypeStruct((E,), _f32),
        jax.ShapeDtypeStruct((E,), _f32),
        jax.ShapeDtypeStruct((E,), _f32),
        jax.ShapeDtypeStruct((E,), _f32),
        jax.ShapeDtypeStruct((E,), _f32),
        jax.ShapeDtypeStruct((E,), _f32),
        jax.ShapeDtypeStruct((E,), _f32),
    ),
    mesh=_MESH,
    compiler_params=pltpu.CompilerParams(
        needs_layout_passes=False, use_tc_tiling_on_sc=False),
    scratch_types=[
        pltpu.VMEM((K,), _i32),
        pltpu.VMEM((K,), _i32),
        pltpu.VMEM((K3,), _i32),
        pltpu.VMEM((K3,), _i32),
        pltpu.VMEM((K, 8), _f32),
        pltpu.VMEM((K, 8), _f32),
        pltpu.VMEM((K,), _f32),
        pltpu.VMEM((K,), _f32),
        pltpu.VMEM((K,), _f32),
        pltpu.VMEM((K3,), _f32),
        pltpu.VMEM((K3,), _f32),
        pltpu.VMEM((K6,), _f32),
        pltpu.VMEM((K6,), _f32),
        pltpu.VMEM((K,), _f32),
        pltpu.VMEM((K,), _f32),
        pltpu.VMEM((K,), _f32),
        pltpu.VMEM((K,), _f32),
        pltpu.VMEM((K,), _f32),
        pltpu.VMEM((K,), _f32),
        pltpu.VMEM((K,), _f32),
        pltpu.VMEM((K,), _f32),
        pltpu.VMEM_SHARED((N * 3,), _f32),
        pltpu.SemaphoreType.DMA,
        pltpu.SemaphoreType.DMA,
        pltpu.SemaphoreType.DMA,
    ],
)
def _beam_sc(tbl, idxA, idxB, pe, pa, pi, zwords,
             o_part, o_fga, o_fgb, o_fl, o_dl, o_ne, o_mm, o_ve, o_m1,
             o_m2, o_l0, o_c, o_s,
             idxA_v, idxB_v, ia3_v, ib3_v, rA_v, rB_v, pe_v, pa_v, pi_v,
             fga_v, fgb_v, fl_v, dl_v,
             ne_v, mm_v, ve_v, m1_v, m2_v, l0_v, c_v, s_v,
             acc, semA, semB, semO):
    cid = lax.axis_index("c")
    sid = lax.axis_index("s")
    wid = cid * NS + sid
    w0 = sid * (ROWS_T * 3)

    @pl.when(sid < NS - 1)
    def _():
        pltpu.sync_copy(zwords.at[pl.ds(0, ROWS_T * 3)],
                        acc.at[pl.ds(w0, ROWS_T * 3)])

    @pl.when(sid == NS - 1)
    def _():
        pltpu.sync_copy(zwords.at[pl.ds(0, ROWS_LAST * 3)],
                        acc.at[pl.ds((NS - 1) * ROWS_T * 3, ROWS_LAST * 3)])

    plsc.subcore_barrier()

    iota = lax.iota(_i32, L)
    cols = [jnp.full((L,), j, _i32) for j in range(6)]

    @pl.loop(0, NB)
    def _block(bi):
        base = wid * EW + bi * K
        pltpu.sync_copy(idxA.at[pl.ds(base, K)], idxA_v)
        pltpu.sync_copy(idxB.at[pl.ds(base, K)], idxB_v)
        ga = pltpu.async_copy(tbl.at[idxA_v], rA_v, semA)
        gb = pltpu.async_copy(tbl.at[idxB_v], rB_v, semB)
        pltpu.sync_copy(pe.at[pl.ds(base, K)], pe_v)
        pltpu.sync_copy(pa.at[pl.ds(base, K)], pa_v)
        pltpu.sync_copy(pi.at[pl.ds(base, K)], pi_v)
        ga.wait()
        gb.wait()

        @pl.loop(0, G)
        def _grp(g):
            off = g * L
            rid = off + iota
            rid3 = rid * 3
            uxA = plsc.load_gather(rA_v, [rid, cols[0]])
            uzA = plsc.load_gather(rA_v, [rid, cols[1]])
            thA = plsc.load_gather(rA_v, [rid, cols[2]])
            xA = plsc.load_gather(rA_v, [rid, cols[3]])
            zA = plsc.load_gather(rA_v, [rid, cols[4]])
            uxB = plsc.load_gather(rB_v, [rid, cols[0]])
            uzB = plsc.load_gather(rB_v, [rid, cols[1]])
            thB = plsc.load_gather(rB_v, [rid, cols[2]])
            xB = plsc.load_gather(rB_v, [rid, cols[3]])
            zB = plsc.load_gather(rB_v, [rid, cols[4]])
            pE = pe_v[pl.ds(off, L)]
            pA = pa_v[pl.ds(off, L)]
            pI = pi_v[pl.ds(off, L)]
            na = idxA_v[pl.ds(off, L)]
            nb = idxB_v[pl.ds(off, L)]

            dx = xB - xA
            dz = zB - zA
            d2 = dx * dx + dz * dz
            r = _rsqrt(d2)
            cc = dx * r
            ss = dz * r
            l0 = d2 * r
            r2 = r * r
            EA = pE * pA
            EI = pE * pI
            EAr = EA * r
            EIr = EI * r
            EIr2 = EI * r2
            EIr3 = EIr * r2

            ua = cc * uxA + ss * uzA
            wa = cc * uzA - ss * uxA
            ub = cc * uxB + ss * uzB
            wb = cc * uzB - ss * uxB
            du = ua - ub
            dw = wa - wb
            tsum = thA + thB

            f0 = EAr * du
            f1 = 12.0 * (EIr3 * dw) + 6.0 * (EIr2 * tsum)
            cdw = 6.0 * (EIr2 * dw)
            f2 = cdw + EIr * (4.0 * thA + 2.0 * thB)
            f5 = cdw + EIr * (2.0 * thA + 4.0 * thB)
            f3 = -f0
            f4 = -f1
            FxA = cc * f0 - ss * f1
            FzA = ss * f0 + cc * f1
            mm = (f5 - f2) * 0.5

            a3 = na * 3
            b3 = nb * 3
            plsc.store_scatter(fga_v, [rid3], FxA)
            plsc.store_scatter(fga_v, [rid3 + 1], FzA)
            plsc.store_scatter(fga_v, [rid3 + 2], f2)
            plsc.store_scatter(fgb_v, [rid3], -FxA)
            plsc.store_scatter(fgb_v, [rid3 + 1], -FzA)
            plsc.store_scatter(fgb_v, [rid3 + 2], f5)
            plsc.store_scatter(ia3_v, [rid3], a3)
            plsc.store_scatter(ia3_v, [rid3 + 1], a3 + 1)
            plsc.store_scatter(ia3_v, [rid3 + 2], a3 + 2)
            plsc.store_scatter(ib3_v, [rid3], b3)
            plsc.store_scatter(ib3_v, [rid3 + 1], b3 + 1)
            plsc.store_scatter(ib3_v, [rid3 + 2], b3 + 2)
            rid6 = rid3 + rid3
            plsc.store_scatter(fl_v, [rid6], f0)
            plsc.store_scatter(fl_v, [rid6 + 1], f1)
            plsc.store_scatter(fl_v, [rid6 + 2], f2)
            plsc.store_scatter(fl_v, [rid6 + 3], f3)
            plsc.store_scatter(fl_v, [rid6 + 4], f4)
            plsc.store_scatter(fl_v, [rid6 + 5], f5)
            plsc.store_scatter(dl_v, [rid6], ua)
            plsc.store_scatter(dl_v, [rid6 + 1], wa)
            plsc.store_scatter(dl_v, [rid6 + 2], thA)
            plsc.store_scatter(dl_v, [rid6 + 3], ub)
            plsc.store_scatter(dl_v, [rid6 + 4], wb)
            plsc.store_scatter(dl_v, [rid6 + 5], thB)
            ne_v[pl.ds(off, L)] = f3
            mm_v[pl.ds(off, L)] = mm
            ve_v[pl.ds(off, L)] = f4
            m1_v[pl.ds(off, L)] = f2
            m2_v[pl.ds(off, L)] = f5
            l0_v[pl.ds(off, L)] = l0
            c_v[pl.ds(off, L)] = cc
            s_v[pl.ds(off, L)] = ss

        outs = [
            pltpu.async_copy(fga_v, o_fga.at[pl.ds(base * 3, K3)], semO),
            pltpu.async_copy(fgb_v, o_fgb.at[pl.ds(base * 3, K3)], semO),
            pltpu.async_copy(fl_v, o_fl.at[pl.ds(base * 6, K6)], semO),
            pltpu.async_copy(dl_v, o_dl.at[pl.ds(base * 6, K6)], semO),
            pltpu.async_copy(ne_v, o_ne.at[pl.ds(base, K)], semO),
            pltpu.async_copy(mm_v, o_mm.at[pl.ds(base, K)], semO),
            pltpu.async_copy(ve_v, o_ve.at[pl.ds(base, K)], semO),
            pltpu.async_copy(m1_v, o_m1.at[pl.ds(base, K)], semO),
            pltpu.async_copy(m2_v, o_m2.at[pl.ds(base, K)], semO),
            pltpu.async_copy(l0_v, o_l0.at[pl.ds(base, K)], semO),
            pltpu.async_copy(c_v, o_c.at[pl.ds(base, K)], semO),
            pltpu.async_copy(s_v, o_s.at[pl.ds(base, K)], semO),
        ]
        pltpu.sync_copy(fga_v, acc.at[ia3_v], add=True)
        pltpu.sync_copy(fgb_v, acc.at[ib3_v], add=True)
        for d in outs:
            d.wait()

    plsc.subcore_barrier()

    @pl.when(sid < NS - 1)
    def _():
        pltpu.sync_copy(acc.at[pl.ds(w0, ROWS_T * 3)],
                        o_part.at[cid, pl.ds(w0, ROWS_T * 3)])

    @pl.when(sid == NS - 1)
    def _():
        pltpu.sync_copy(
            acc.at[pl.ds((NS - 1) * ROWS_T * 3, ROWS_LAST * 3)],
            o_part.at[cid, pl.ds((NS - 1) * ROWS_T * 3, ROWS_LAST * 3)])


def kernel(pred_disp, coords, prop_E, prop_A, prop_I22, connectivity):
    tbl = jnp.concatenate(
        [pred_disp, coords[:, 0:1], coords[:, 2:3],
         jnp.zeros((N, 3), _f32)], axis=1)
    idxA = connectivity[:, 0].astype(_i32)
    idxB = connectivity[:, 1].astype(_i32)
    zwords = jnp.zeros((ROWS_T * 3,), _f32)
    (part, fga, fgb, fl, dl, ne, mm, ve, m1, m2, l0, c, s) = _beam_sc(
        tbl, idxA, idxB, prop_E, prop_A, prop_I22, zwords)
    nodal = (part[0] + part[1]).reshape(N, 3)
    return (nodal, fga.reshape(E, 3), fgb.reshape(E, 3),
            fl.reshape(E, 6), dl.reshape(E, 6),
            ne, mm, ve, m1, m2, l0, c, s)


# speedup vs baseline: 4.6400x; 4.6400x over previous
import functools

import jax
import jax.numpy as jnp
from jax import lax
from jax.experimental import pallas as pl
from jax.experimental.pallas import tpu as pltpu
from jax.experimental.pallas import tpu_sc as plsc

N = 100000
E = 6400000
NC, NS, L = 2, 16, 16
NW = NC * NS
EW = E // NW
K = 800
NB = EW // K
G = K // L
ROWS_T = 6256
ROWS_LAST = N - (NS - 1) * ROWS_T

_MESH = plsc.VectorSubcoreMesh(
    core_axis_name="c", subcore_axis_name="s", num_cores=NC, num_subcores=NS)

_f32 = jnp.float32
_i32 = jnp.int32

_EDGE_COL = jax.ShapeDtypeStruct((E,), _f32)


def _rsqrt(d2):
    i = plsc.bitcast(d2, _i32)
    i = jnp.int32(0x5F3759DF) - lax.shift_right_logical(i, 1)
    r = plsc.bitcast(i, _f32)
    hd = 0.5 * d2
    for _ in range(3):
        r = r * (1.5 - hd * r * r)
    return r


@functools.partial(
    pl.kernel,
    out_type=(
        (jax.ShapeDtypeStruct((NC, N * 3), _f32),)
        + (_EDGE_COL,) * 15
    ),
    mesh=_MESH,
    compiler_params=pltpu.CompilerParams(
        needs_layout_passes=False, use_tc_tiling_on_sc=False),
    scratch_types=(
        [pltpu.VMEM((K,), _i32)] * 2
        + [pltpu.VMEM((K, 8), _f32)] * 2
        + [pltpu.VMEM((K,), _f32)] * 3
        + [pltpu.VMEM((K,), _f32)] * 15
        + [pltpu.VMEM((K,), _f32)] * 2
        + [pltpu.VMEM((K,), _i32)] * 6
        + [
            pltpu.VMEM_SHARED((N * 3,), _f32),
            pltpu.SemaphoreType.DMA,
            pltpu.SemaphoreType.DMA,
        ]
    ),
)
def _beam_sc(tbl, idxA, idxB, pe, pa, pi, zwords,
             o_part, o_f0, o_f1, o_f2, o_f5,
             o_ua, o_wa, o_ta, o_ub, o_wb, o_tb,
             o_fxA, o_fzA, o_l0, o_c, o_s,
             idxA_v, idxB_v, rA_v, rB_v, pe_v, pa_v, pi_v,
             f0_v, f1_v, f2_v, f5_v,
             ua_v, wa_v, ta_v, ub_v, wb_v, tb_v,
             fxA_v, fzA_v, l0_v, c_v, s_v,
             fxB_v, fzB_v,
             ia0_v, ia1_v, ia2_v, ib0_v, ib1_v, ib2_v,
             acc, semG, semO):
    cid = lax.axis_index("c")
    sid = lax.axis_index("s")
    wid = cid * NS + sid
    w0 = sid * (ROWS_T * 3)

    @pl.when(sid < NS - 1)
    def _():
        pltpu.sync_copy(zwords.at[pl.ds(0, ROWS_T * 3)],
                        acc.at[pl.ds(w0, ROWS_T * 3)])

    @pl.when(sid == NS - 1)
    def _():
        pltpu.sync_copy(zwords.at[pl.ds(0, ROWS_LAST * 3)],
                        acc.at[pl.ds((NS - 1) * ROWS_T * 3, ROWS_LAST * 3)])

    plsc.subcore_barrier()

    iota = lax.iota(_i32, L)
    cols = [jnp.full((L,), j, _i32) for j in range(5)]

    @pl.loop(0, NB)
    def _block(bi):
        base = wid * EW + bi * K
        pltpu.sync_copy(idxA.at[pl.ds(base, K)], idxA_v)
        pltpu.sync_copy(idxB.at[pl.ds(base, K)], idxB_v)
        ga = pltpu.async_copy(tbl.at[idxA_v], rA_v, semG)
        gb = pltpu.async_copy(tbl.at[idxB_v], rB_v, semG)
        pltpu.sync_copy(pe.at[pl.ds(base, K)], pe_v)
        pltpu.sync_copy(pa.at[pl.ds(base, K)], pa_v)
        pltpu.sync_copy(pi.at[pl.ds(base, K)], pi_v)
        ga.wait()
        gb.wait()

        @pl.loop(0, G)
        def _grp(g):
            off = g * L
            rid = off + iota
            uxA = plsc.load_gather(rA_v, [rid, cols[0]])
            uzA = plsc.load_gather(rA_v, [rid, cols[1]])
            thA = plsc.load_gather(rA_v, [rid, cols[2]])
            xA = plsc.load_gather(rA_v, [rid, cols[3]])
            zA = plsc.load_gather(rA_v, [rid, cols[4]])
            uxB = plsc.load_gather(rB_v, [rid, cols[0]])
            uzB = plsc.load_gather(rB_v, [rid, cols[1]])
            thB = plsc.load_gather(rB_v, [rid, cols[2]])
            xB = plsc.load_gather(rB_v, [rid, cols[3]])
            zB = plsc.load_gather(rB_v, [rid, cols[4]])
            pE = pe_v[pl.ds(off, L)]
            pA = pa_v[pl.ds(off, L)]
            pI = pi_v[pl.ds(off, L)]
            na = idxA_v[pl.ds(off, L)]
            nb = idxB_v[pl.ds(off, L)]

            dx = xB - xA
            dz = zB - zA
            d2 = dx * dx + dz * dz
            r = _rsqrt(d2)
            cc = dx * r
            ss = dz * r
            l0 = d2 * r
            r2 = r * r
            EA = pE * pA
            EI = pE * pI
            EAr = EA * r
            EIr = EI * r
            EIr2 = EI * r2
            EIr3 = EIr * r2

            ua = cc * uxA + ss * uzA
            wa = cc * uzA - ss * uxA
            ub = cc * uxB + ss * uzB
            wb = cc * uzB - ss * uxB
            du = ua - ub
            dw = wa - wb
            tsum = thA + thB

            f0 = EAr * du
            f1 = 12.0 * (EIr3 * dw) + 6.0 * (EIr2 * tsum)
            cdw = 6.0 * (EIr2 * dw)
            f2 = cdw + EIr * (4.0 * thA + 2.0 * thB)
            f5 = cdw + EIr * (2.0 * thA + 4.0 * thB)
            FxA = cc * f0 - ss * f1
            FzA = ss * f0 + cc * f1

            a3 = na * 3
            b3 = nb * 3
            s_ = pl.ds(off, L)
            f0_v[s_] = f0
            f1_v[s_] = f1
            f2_v[s_] = f2
            f5_v[s_] = f5
            ua_v[s_] = ua
            wa_v[s_] = wa
            ta_v[s_] = thA
            ub_v[s_] = ub
            wb_v[s_] = wb
            tb_v[s_] = thB
            fxA_v[s_] = FxA
            fzA_v[s_] = FzA
            l0_v[s_] = l0
            c_v[s_] = cc
            s_v[s_] = ss
            fxB_v[s_] = -FxA
            fzB_v[s_] = -FzA
            ia0_v[s_] = a3
            ia1_v[s_] = a3 + 1
            ia2_v[s_] = a3 + 2
            ib0_v[s_] = b3
            ib1_v[s_] = b3 + 1
            ib2_v[s_] = b3 + 2

        sl = pl.ds(base, K)
        outs = [
            pltpu.async_copy(f0_v, o_f0.at[sl], semO),
            pltpu.async_copy(f1_v, o_f1.at[sl], semO),
            pltpu.async_copy(f2_v, o_f2.at[sl], semO),
            pltpu.async_copy(f5_v, o_f5.at[sl], semO),
            pltpu.async_copy(ua_v, o_ua.at[sl], semO),
            pltpu.async_copy(wa_v, o_wa.at[sl], semO),
            pltpu.async_copy(ta_v, o_ta.at[sl], semO),
            pltpu.async_copy(ub_v, o_ub.at[sl], semO),
            pltpu.async_copy(wb_v, o_wb.at[sl], semO),
            pltpu.async_copy(tb_v, o_tb.at[sl], semO),
            pltpu.async_copy(fxA_v, o_fxA.at[sl], semO),
            pltpu.async_copy(fzA_v, o_fzA.at[sl], semO),
            pltpu.async_copy(l0_v, o_l0.at[sl], semO),
            pltpu.async_copy(c_v, o_c.at[sl], semO),
            pltpu.async_copy(s_v, o_s.at[sl], semO),
        ]
        pltpu.sync_copy(fxA_v, acc.at[ia0_v], add=True)
        pltpu.sync_copy(fzA_v, acc.at[ia1_v], add=True)
        pltpu.sync_copy(f2_v, acc.at[ia2_v], add=True)
        pltpu.sync_copy(fxB_v, acc.at[ib0_v], add=True)
        pltpu.sync_copy(fzB_v, acc.at[ib1_v], add=True)
        pltpu.sync_copy(f5_v, acc.at[ib2_v], add=True)
        for d in outs:
            d.wait()

    plsc.subcore_barrier()

    @pl.when(sid < NS - 1)
    def _():
        pltpu.sync_copy(acc.at[pl.ds(w0, ROWS_T * 3)],
                        o_part.at[cid, pl.ds(w0, ROWS_T * 3)])

    @pl.when(sid == NS - 1)
    def _():
        pltpu.sync_copy(
            acc.at[pl.ds((NS - 1) * ROWS_T * 3, ROWS_LAST * 3)],
            o_part.at[cid, pl.ds((NS - 1) * ROWS_T * 3, ROWS_LAST * 3)])


def kernel(pred_disp, coords, prop_E, prop_A, prop_I22, connectivity):
    tbl = jnp.concatenate(
        [pred_disp, coords[:, 0:1], coords[:, 2:3],
         jnp.zeros((N, 3), _f32)], axis=1)
    idxA = connectivity[:, 0].astype(_i32)
    idxB = connectivity[:, 1].astype(_i32)
    zwords = jnp.zeros((ROWS_T * 3,), _f32)
    (part, f0, f1, f2, f5, ua, wa, ta, ub, wb, tb,
     fxA, fzA, l0, c, s) = _beam_sc(
        tbl, idxA, idxB, prop_E, prop_A, prop_I22, zwords)
    nodal = (part[0] + part[1]).reshape(N, 3)
    fga = jnp.stack([fxA, fzA, f2], axis=1)
    fgb = jnp.stack([-fxA, -fzA, f5], axis=1)
    fl = jnp.stack([f0, f1, f2, -f0, -f1, f5], axis=1)
    dl = jnp.stack([ua, wa, ta, ub, wb, tb], axis=1)
    ne = -f0
    ve = -f1
    mm = (f5 - f2) * 0.5
    return (nodal, fga, fgb, fl, dl, ne, mm, ve, f2, f5, l0, c, s)


# speedup vs baseline: 5.1125x; 1.1018x over previous
import functools

import jax
import jax.numpy as jnp
from jax import lax
from jax.experimental import pallas as pl
from jax.experimental.pallas import tpu as pltpu
from jax.experimental.pallas import tpu_sc as plsc

N = 100000
E = 6400000
NC, NS, L = 2, 16, 16
NW = NC * NS
EW = E // NW
K = 2000
NB = EW // K
G = K // L
ROWS_T = 6256
ROWS_LAST = N - (NS - 1) * ROWS_T

_MESH = plsc.VectorSubcoreMesh(
    core_axis_name="c", subcore_axis_name="s", num_cores=NC, num_subcores=NS)

_f32 = jnp.float32
_i32 = jnp.int32

_EDGE_COL = jax.ShapeDtypeStruct((E,), _f32)


def _rsqrt(d2):
    i = plsc.bitcast(d2, _i32)
    i = jnp.int32(0x5F3759DF) - lax.shift_right_logical(i, 1)
    r = plsc.bitcast(i, _f32)
    hd = 0.5 * d2
    for _ in range(3):
        r = r * (1.5 - hd * r * r)
    return r


@functools.partial(
    pl.kernel,
    out_type=(
        (jax.ShapeDtypeStruct((NC, N * 3), _f32),)
        + (_EDGE_COL,) * 15
    ),
    mesh=_MESH,
    compiler_params=pltpu.CompilerParams(
        needs_layout_passes=False, use_tc_tiling_on_sc=False),
    scratch_types=(
        [pltpu.VMEM((K,), _i32)] * 2
        + [pltpu.VMEM((K, 8), _f32)] * 2
        + [pltpu.VMEM((K,), _f32)] * 3
        + [pltpu.VMEM((K,), _f32)] * 15
        + [pltpu.VMEM((K,), _f32)] * 2
        + [pltpu.VMEM((K,), _i32)] * 6
        + [
            pltpu.VMEM_SHARED((N * 3,), _f32),
            pltpu.SemaphoreType.DMA,
            pltpu.SemaphoreType.DMA,
        ]
    ),
)
def _beam_sc(tbl, idxA, idxB, pe, pa, pi, zwords,
             o_part, o_f0, o_f1, o_f2, o_f5,
             o_ua, o_wa, o_ta, o_ub, o_wb, o_tb,
             o_fxA, o_fzA, o_l0, o_c, o_s,
             idxA_v, idxB_v, rA_v, rB_v, pe_v, pa_v, pi_v,
             f0_v, f1_v, f2_v, f5_v,
             ua_v, wa_v, ta_v, ub_v, wb_v, tb_v,
             fxA_v, fzA_v, l0_v, c_v, s_v,
             fxB_v, fzB_v,
             ia0_v, ia1_v, ia2_v, ib0_v, ib1_v, ib2_v,
             acc, semG, semO):
    cid = lax.axis_index("c")
    sid = lax.axis_index("s")
    wid = cid * NS + sid
    w0 = sid * (ROWS_T * 3)

    @pl.when(sid < NS - 1)
    def _():
        pltpu.sync_copy(zwords.at[pl.ds(0, ROWS_T * 3)],
                        acc.at[pl.ds(w0, ROWS_T * 3)])

    @pl.when(sid == NS - 1)
    def _():
        pltpu.sync_copy(zwords.at[pl.ds(0, ROWS_LAST * 3)],
                        acc.at[pl.ds((NS - 1) * ROWS_T * 3, ROWS_LAST * 3)])

    plsc.subcore_barrier()

    iota = lax.iota(_i32, L)
    cols = [jnp.full((L,), j, _i32) for j in range(5)]

    @pl.loop(0, NB)
    def _block(bi):
        base = wid * EW + bi * K
        pltpu.sync_copy(idxA.at[pl.ds(base, K)], idxA_v)
        pltpu.sync_copy(idxB.at[pl.ds(base, K)], idxB_v)
        ga = pltpu.async_copy(tbl.at[idxA_v], rA_v, semG)
        gb = pltpu.async_copy(tbl.at[idxB_v], rB_v, semG)
        pltpu.sync_copy(pe.at[pl.ds(base, K)], pe_v)
        pltpu.sync_copy(pa.at[pl.ds(base, K)], pa_v)
        pltpu.sync_copy(pi.at[pl.ds(base, K)], pi_v)
        ga.wait()
        gb.wait()

        @pl.loop(0, G)
        def _grp(g):
            off = g * L
            rid = off + iota
            uxA = plsc.load_gather(rA_v, [rid, cols[0]])
            uzA = plsc.load_gather(rA_v, [rid, cols[1]])
            thA = plsc.load_gather(rA_v, [rid, cols[2]])
            xA = plsc.load_gather(rA_v, [rid, cols[3]])
            zA = plsc.load_gather(rA_v, [rid, cols[4]])
            uxB = plsc.load_gather(rB_v, [rid, cols[0]])
            uzB = plsc.load_gather(rB_v, [rid, cols[1]])
            thB = plsc.load_gather(rB_v, [rid, cols[2]])
            xB = plsc.load_gather(rB_v, [rid, cols[3]])
            zB = plsc.load_gather(rB_v, [rid, cols[4]])
            pE = pe_v[pl.ds(off, L)]
            pA = pa_v[pl.ds(off, L)]
            pI = pi_v[pl.ds(off, L)]
            na = idxA_v[pl.ds(off, L)]
            nb = idxB_v[pl.ds(off, L)]

            dx = xB - xA
            dz = zB - zA
            d2 = dx * dx + dz * dz
            r = _rsqrt(d2)
            cc = dx * r
            ss = dz * r
            l0 = d2 * r
            r2 = r * r
            EA = pE * pA
            EI = pE * pI
            EAr = EA * r
            EIr = EI * r
            EIr2 = EI * r2
            EIr3 = EIr * r2

            ua = cc * uxA + ss * uzA
            wa = cc * uzA - ss * uxA
            ub = cc * uxB + ss * uzB
            wb = cc * uzB - ss * uxB
            du = ua - ub
            dw = wa - wb
            tsum = thA + thB

            f0 = EAr * du
            f1 = 12.0 * (EIr3 * dw) + 6.0 * (EIr2 * tsum)
            cdw = 6.0 * (EIr2 * dw)
            f2 = cdw + EIr * (4.0 * thA + 2.0 * thB)
            f5 = cdw + EIr * (2.0 * thA + 4.0 * thB)
            FxA = cc * f0 - ss * f1
            FzA = ss * f0 + cc * f1

            a3 = na * 3
            b3 = nb * 3
            s_ = pl.ds(off, L)
            f0_v[s_] = f0
            f1_v[s_] = f1
            f2_v[s_] = f2
            f5_v[s_] = f5
            ua_v[s_] = ua
            wa_v[s_] = wa
            ta_v[s_] = thA
            ub_v[s_] = ub
            wb_v[s_] = wb
            tb_v[s_] = thB
            fxA_v[s_] = FxA
            fzA_v[s_] = FzA
            l0_v[s_] = l0
            c_v[s_] = cc
            s_v[s_] = ss
            fxB_v[s_] = -FxA
            fzB_v[s_] = -FzA
            ia0_v[s_] = a3
            ia1_v[s_] = a3 + 1
            ia2_v[s_] = a3 + 2
            ib0_v[s_] = b3
            ib1_v[s_] = b3 + 1
            ib2_v[s_] = b3 + 2

        sl = pl.ds(base, K)
        outs = [
            pltpu.async_copy(f0_v, o_f0.at[sl], semO),
            pltpu.async_copy(f1_v, o_f1.at[sl], semO),
            pltpu.async_copy(f2_v, o_f2.at[sl], semO),
            pltpu.async_copy(f5_v, o_f5.at[sl], semO),
            pltpu.async_copy(ua_v, o_ua.at[sl], semO),
            pltpu.async_copy(wa_v, o_wa.at[sl], semO),
            pltpu.async_copy(ta_v, o_ta.at[sl], semO),
            pltpu.async_copy(ub_v, o_ub.at[sl], semO),
            pltpu.async_copy(wb_v, o_wb.at[sl], semO),
            pltpu.async_copy(tb_v, o_tb.at[sl], semO),
            pltpu.async_copy(fxA_v, o_fxA.at[sl], semO),
            pltpu.async_copy(fzA_v, o_fzA.at[sl], semO),
            pltpu.async_copy(l0_v, o_l0.at[sl], semO),
            pltpu.async_copy(c_v, o_c.at[sl], semO),
            pltpu.async_copy(s_v, o_s.at[sl], semO),
        ]
        pltpu.sync_copy(fxA_v, acc.at[ia0_v], add=True)
        pltpu.sync_copy(fzA_v, acc.at[ia1_v], add=True)
        pltpu.sync_copy(f2_v, acc.at[ia2_v], add=True)
        pltpu.sync_copy(fxB_v, acc.at[ib0_v], add=True)
        pltpu.sync_copy(fzB_v, acc.at[ib1_v], add=True)
        pltpu.sync_copy(f5_v, acc.at[ib2_v], add=True)
        for d in outs:
            d.wait()

    plsc.subcore_barrier()

    @pl.when(sid < NS - 1)
    def _():
        pltpu.sync_copy(acc.at[pl.ds(w0, ROWS_T * 3)],
                        o_part.at[cid, pl.ds(w0, ROWS_T * 3)])

    @pl.when(sid == NS - 1)
    def _():
        pltpu.sync_copy(
            acc.at[pl.ds((NS - 1) * ROWS_T * 3, ROWS_LAST * 3)],
            o_part.at[cid, pl.ds((NS - 1) * ROWS_T * 3, ROWS_LAST * 3)])


def kernel(pred_disp, coords, prop_E, prop_A, prop_I22, connectivity):
    tbl = jnp.concatenate(
        [pred_disp, coords[:, 0:1], coords[:, 2:3],
         jnp.zeros((N, 3), _f32)], axis=1)
    idxA = connectivity[:, 0].astype(_i32)
    idxB = connectivity[:, 1].astype(_i32)
    zwords = jnp.zeros((ROWS_T * 3,), _f32)
    (part, f0, f1, f2, f5, ua, wa, ta, ub, wb, tb,
     fxA, fzA, l0, c, s) = _beam_sc(
        tbl, idxA, idxB, prop_E, prop_A, prop_I22, zwords)
    nodal = (part[0] + part[1]).reshape(N, 3)
    fga = jnp.stack([fxA, fzA, f2], axis=1)
    fgb = jnp.stack([-fxA, -fzA, f5], axis=1)
    fl = jnp.stack([f0, f1, f2, -f0, -f1, f5], axis=1)
    dl = jnp.stack([ua, wa, ta, ub, wb, tb], axis=1)
    ne = -f0
    ve = -f1
    mm = (f5 - f2) * 0.5
    return (nodal, fga, fgb, fl, dl, ne, mm, ve, f2, f5, l0, c, s)


# speedup vs baseline: 5.5723x; 1.0899x over previous
import functools

import jax
import jax.numpy as jnp
from jax import lax
from jax.experimental import pallas as pl
from jax.experimental.pallas import tpu as pltpu
from jax.experimental.pallas import tpu_sc as plsc

N = 100000
E = 6400000
NC, NS, L = 2, 16, 16
NW = NC * NS
EW = E // NW
K = 800
K3 = 3 * K
NB = EW // K
G = K // L
ROWS_T = 6256
ROWS_LAST = N - (NS - 1) * ROWS_T

_MESH = plsc.VectorSubcoreMesh(
    core_axis_name="c", subcore_axis_name="s", num_cores=NC, num_subcores=NS)

_f32 = jnp.float32
_i32 = jnp.int32

_EDGE_COL = jax.ShapeDtypeStruct((E,), _f32)

_IN_SET = [pltpu.VMEM((K,), _i32)] * 2 + [pltpu.VMEM((K, 8), _f32)] * 2 + \
    [pltpu.VMEM((K,), _f32)] * 3


def _rsqrt(d2):
    i = plsc.bitcast(d2, _i32)
    i = jnp.int32(0x5F3759DF) - lax.shift_right_logical(i, 1)
    r = plsc.bitcast(i, _f32)
    hd = 0.5 * d2
    for _ in range(3):
        r = r * (1.5 - hd * r * r)
    return r


@functools.partial(
    pl.kernel,
    out_type=(
        (jax.ShapeDtypeStruct((NC, N * 3), _f32),)
        + (_EDGE_COL,) * 15
    ),
    mesh=_MESH,
    compiler_params=pltpu.CompilerParams(
        needs_layout_passes=False, use_tc_tiling_on_sc=False),
    scratch_types=(
        _IN_SET + _IN_SET
        + [pltpu.VMEM((K,), _f32)] * 15
        + [
            pltpu.VMEM((2 * K3,), _f32),
            pltpu.VMEM((2 * K3,), _i32),
            pltpu.VMEM_SHARED((N * 3,), _f32),
            pltpu.SemaphoreType.DMA,
            pltpu.SemaphoreType.DMA,
        ]
    ),
)
def _beam_sc(tbl, idxA, idxB, pe, pa, pi, zwords,
             o_part, o_f0, o_f1, o_f2, o_f5,
             o_ua, o_wa, o_ta, o_ub, o_wb, o_tb,
             o_fxA, o_fzA, o_l0, o_c, o_s,
             iA0, iB0, rA0, rB0, pe0, pa0, pi0,
             iA1, iB1, rA1, rB1, pe1, pa1, pi1,
             f0_v, f1_v, f2_v, f5_v,
             ua_v, wa_v, ta_v, ub_v, wb_v, tb_v,
             fxA_v, fzA_v, l0_v, c_v, s_v,
             fgab_v, iab3_v,
             acc, semI, semO):
    cid = lax.axis_index("c")
    sid = lax.axis_index("s")
    wid = cid * NS + sid
    w0 = sid * (ROWS_T * 3)
    sets = ((iA0, iB0, rA0, rB0, pe0, pa0, pi0),
            (iA1, iB1, rA1, rB1, pe1, pa1, pi1))

    @pl.when(sid < NS - 1)
    def _():
        pltpu.sync_copy(zwords.at[pl.ds(0, ROWS_T * 3)],
                        acc.at[pl.ds(w0, ROWS_T * 3)])

    @pl.when(sid == NS - 1)
    def _():
        pltpu.sync_copy(zwords.at[pl.ds(0, ROWS_LAST * 3)],
                        acc.at[pl.ds((NS - 1) * ROWS_T * 3, ROWS_LAST * 3)])

    plsc.subcore_barrier()

    iota = lax.iota(_i32, L)
    cols = [jnp.full((L,), j, _i32) for j in range(5)]

    def issue_inputs(blk, st):
        iA_v, iB_v, rA_v, rB_v, pe_v, pa_v, pi_v = st
        b = wid * EW + blk * K
        pltpu.sync_copy(idxA.at[pl.ds(b, K)], iA_v)
        pltpu.sync_copy(idxB.at[pl.ds(b, K)], iB_v)
        pltpu.async_copy(tbl.at[iA_v], rA_v, semI)
        pltpu.async_copy(tbl.at[iB_v], rB_v, semI)
        pltpu.async_copy(pe.at[pl.ds(b, K)], pe_v, semI)
        pltpu.async_copy(pa.at[pl.ds(b, K)], pa_v, semI)
        pltpu.async_copy(pi.at[pl.ds(b, K)], pi_v, semI)

    def drain_inputs(st):
        _, _, rA_v, rB_v, pe_v, pa_v, pi_v = st
        pltpu.make_async_copy(tbl.at[pl.ds(0, K)], rA_v, semI).wait()
        pltpu.make_async_copy(tbl.at[pl.ds(0, K)], rB_v, semI).wait()
        pltpu.make_async_copy(pe.at[pl.ds(0, K)], pe_v, semI).wait()
        pltpu.make_async_copy(pa.at[pl.ds(0, K)], pa_v, semI).wait()
        pltpu.make_async_copy(pi.at[pl.ds(0, K)], pi_v, semI).wait()

    def compute_block(blk, st):
        iA_v, iB_v, rA_v, rB_v, pe_v, pa_v, pi_v = st

        @pl.loop(0, G)
        def _grp(g):
            off = g * L
            rid = off + iota
            rid3 = rid * 3
            uxA = plsc.load_gather(rA_v, [rid, cols[0]])
            uzA = plsc.load_gather(rA_v, [rid, cols[1]])
            thA = plsc.load_gather(rA_v, [rid, cols[2]])
            xA = plsc.load_gather(rA_v, [rid, cols[3]])
            zA = plsc.load_gather(rA_v, [rid, cols[4]])
            uxB = plsc.load_gather(rB_v, [rid, cols[0]])
            uzB = plsc.load_gather(rB_v, [rid, cols[1]])
            thB = plsc.load_gather(rB_v, [rid, cols[2]])
            xB = plsc.load_gather(rB_v, [rid, cols[3]])
            zB = plsc.load_gather(rB_v, [rid, cols[4]])
            pE = pe_v[pl.ds(off, L)]
            pA = pa_v[pl.ds(off, L)]
            pI = pi_v[pl.ds(off, L)]
            na = iA_v[pl.ds(off, L)]
            nb = iB_v[pl.ds(off, L)]

            dx = xB - xA
            dz = zB - zA
            d2 = dx * dx + dz * dz
            r = _rsqrt(d2)
            cc = dx * r
            ss = dz * r
            l0 = d2 * r
            r2 = r * r
            EA = pE * pA
            EI = pE * pI
            EAr = EA * r
            EIr = EI * r
            EIr2 = EI * r2
            EIr3 = EIr * r2

            ua = cc * uxA + ss * uzA
            wa = cc * uzA - ss * uxA
            ub = cc * uxB + ss * uzB
            wb = cc * uzB - ss * uxB
            du = ua - ub
            dw = wa - wb
            tsum = thA + thB

            f0 = EAr * du
            f1 = 12.0 * (EIr3 * dw) + 6.0 * (EIr2 * tsum)
            cdw = 6.0 * (EIr2 * dw)
            f2 = cdw + EIr * (4.0 * thA + 2.0 * thB)
            f5 = cdw + EIr * (2.0 * thA + 4.0 * thB)
            FxA = cc * f0 - ss * f1
            FzA = ss * f0 + cc * f1

            a3 = na * 3
            b3 = nb * 3
            s_ = pl.ds(off, L)
            f0_v[s_] = f0
            f1_v[s_] = f1
            f2_v[s_] = f2
            f5_v[s_] = f5
            ua_v[s_] = ua
            wa_v[s_] = wa
            ta_v[s_] = thA
            ub_v[s_] = ub
            wb_v[s_] = wb
            tb_v[s_] = thB
            fxA_v[s_] = FxA
            fzA_v[s_] = FzA
            l0_v[s_] = l0
            c_v[s_] = cc
            s_v[s_] = ss
            plsc.store_scatter(fgab_v, [rid3], FxA)
            plsc.store_scatter(fgab_v, [rid3 + 1], FzA)
            plsc.store_scatter(fgab_v, [rid3 + 2], f2)
            plsc.store_scatter(fgab_v, [K3 + rid3], -FxA)
            plsc.store_scatter(fgab_v, [K3 + rid3 + 1], -FzA)
            plsc.store_scatter(fgab_v, [K3 + rid3 + 2], f5)
            plsc.store_scatter(iab3_v, [rid3], a3)
            plsc.store_scatter(iab3_v, [rid3 + 1], a3 + 1)
            plsc.store_scatter(iab3_v, [rid3 + 2], a3 + 2)
            plsc.store_scatter(iab3_v, [K3 + rid3], b3)
            plsc.store_scatter(iab3_v, [K3 + rid3 + 1], b3 + 1)
            plsc.store_scatter(iab3_v, [K3 + rid3 + 2], b3 + 2)

        base = wid * EW + blk * K
        sl = pl.ds(base, K)
        outs = [
            pltpu.async_copy(f0_v, o_f0.at[sl], semO),
            pltpu.async_copy(f1_v, o_f1.at[sl], semO),
            pltpu.async_copy(f2_v, o_f2.at[sl], semO),
            pltpu.async_copy(f5_v, o_f5.at[sl], semO),
            pltpu.async_copy(ua_v, o_ua.at[sl], semO),
            pltpu.async_copy(wa_v, o_wa.at[sl], semO),
            pltpu.async_copy(ta_v, o_ta.at[sl], semO),
            pltpu.async_copy(ub_v, o_ub.at[sl], semO),
            pltpu.async_copy(wb_v, o_wb.at[sl], semO),
            pltpu.async_copy(tb_v, o_tb.at[sl], semO),
            pltpu.async_copy(fxA_v, o_fxA.at[sl], semO),
            pltpu.async_copy(fzA_v, o_fzA.at[sl], semO),
            pltpu.async_copy(l0_v, o_l0.at[sl], semO),
            pltpu.async_copy(c_v, o_c.at[sl], semO),
            pltpu.async_copy(s_v, o_s.at[sl], semO),
        ]
        pltpu.sync_copy(fgab_v, acc.at[iab3_v], add=True)
        for d in outs:
            d.wait()

    issue_inputs(jnp.int32(0), sets[0])

    @pl.loop(0, NB // 2)
    def _pair(j):
        b0 = 2 * j
        drain_inputs(sets[0])
        issue_inputs(b0 + 1, sets[1])
        compute_block(b0, sets[0])
        drain_inputs(sets[1])

        @pl.when(b0 + 2 < NB)
        def _():
            issue_inputs(b0 + 2, sets[0])

        compute_block(b0 + 1, sets[1])

    plsc.subcore_barrier()

    @pl.when(sid < NS - 1)
    def _():
        pltpu.sync_copy(acc.at[pl.ds(w0, ROWS_T * 3)],
                        o_part.at[cid, pl.ds(w0, ROWS_T * 3)])

    @pl.when(sid == NS - 1)
    def _():
        pltpu.sync_copy(
            acc.at[pl.ds((NS - 1) * ROWS_T * 3, ROWS_LAST * 3)],
            o_part.at[cid, pl.ds((NS - 1) * ROWS_T * 3, ROWS_LAST * 3)])


def kernel(pred_disp, coords, prop_E, prop_A, prop_I22, connectivity):
    tbl = jnp.concatenate(
        [pred_disp, coords[:, 0:1], coords[:, 2:3],
         jnp.zeros((N, 3), _f32)], axis=1)
    idxA = connectivity[:, 0].astype(_i32)
    idxB = connectivity[:, 1].astype(_i32)
    zwords = jnp.zeros((ROWS_T * 3,), _f32)
    (part, f0, f1, f2, f5, ua, wa, ta, ub, wb, tb,
     fxA, fzA, l0, c, s) = _beam_sc(
        tbl, idxA, idxB, prop_E, prop_A, prop_I22, zwords)
    nodal = (part[0] + part[1]).reshape(N, 3)
    fga = jnp.stack([fxA, fzA, f2], axis=1)
    fgb = jnp.stack([-fxA, -fzA, f5], axis=1)
    fl = jnp.stack([f0, f1, f2, -f0, -f1, f5], axis=1)
    dl = jnp.stack([ua, wa, ta, ub, wb, tb], axis=1)
    ne = -f0
    ve = -f1
    mm = (f5 - f2) * 0.5
    return (nodal, fga, fgb, fl, dl, ne, mm, ve, f2, f5, l0, c, s)


# speedup vs baseline: 5.8617x; 1.0519x over previous
import functools

import jax
import jax.numpy as jnp
from jax import lax
from jax.experimental import pallas as pl
from jax.experimental.pallas import tpu as pltpu
from jax.experimental.pallas import tpu_sc as plsc

N = 100000
E = 6400000
NC, NS, L = 2, 16, 16
NW = NC * NS
EW = E // NW
K = 1600
K3 = 3 * K
NB = EW // K
G = K // L
ROWS_T = 6256
ROWS_LAST = N - (NS - 1) * ROWS_T

_MESH = plsc.VectorSubcoreMesh(
    core_axis_name="c", subcore_axis_name="s", num_cores=NC, num_subcores=NS)

_f32 = jnp.float32
_i32 = jnp.int32

_EDGE_COL = jax.ShapeDtypeStruct((E,), _f32)

_IN_SET = [pltpu.VMEM((K,), _i32)] * 2 + [pltpu.VMEM((K, 8), _f32)] * 2 + \
    [pltpu.VMEM((K,), _f32)] * 3


def _rsqrt(d2):
    i = plsc.bitcast(d2, _i32)
    i = jnp.int32(0x5F3759DF) - lax.shift_right_logical(i, 1)
    r = plsc.bitcast(i, _f32)
    hd = 0.5 * d2
    for _ in range(3):
        r = r * (1.5 - hd * r * r)
    return r


@functools.partial(
    pl.kernel,
    out_type=(
        (jax.ShapeDtypeStruct((NC, N * 3), _f32),)
        + (_EDGE_COL,) * 15
    ),
    mesh=_MESH,
    compiler_params=pltpu.CompilerParams(
        needs_layout_passes=False, use_tc_tiling_on_sc=False),
    scratch_types=(
        _IN_SET + _IN_SET
        + [pltpu.VMEM((K,), _f32)] * 15
        + [
            pltpu.VMEM((2 * K3,), _f32),
            pltpu.VMEM((2 * K3,), _i32),
            pltpu.VMEM_SHARED((N * 3,), _f32),
            pltpu.SemaphoreType.DMA,
            pltpu.SemaphoreType.DMA,
        ]
    ),
)
def _beam_sc(tbl, idxA, idxB, pe, pa, pi, zwords,
             o_part, o_f0, o_f1, o_f2, o_f5,
             o_ua, o_wa, o_ta, o_ub, o_wb, o_tb,
             o_fxA, o_fzA, o_l0, o_c, o_s,
             iA0, iB0, rA0, rB0, pe0, pa0, pi0,
             iA1, iB1, rA1, rB1, pe1, pa1, pi1,
             f0_v, f1_v, f2_v, f5_v,
             ua_v, wa_v, ta_v, ub_v, wb_v, tb_v,
             fxA_v, fzA_v, l0_v, c_v, s_v,
             fgab_v, iab3_v,
             acc, semI, semO):
    cid = lax.axis_index("c")
    sid = lax.axis_index("s")
    wid = cid * NS + sid
    w0 = sid * (ROWS_T * 3)
    sets = ((iA0, iB0, rA0, rB0, pe0, pa0, pi0),
            (iA1, iB1, rA1, rB1, pe1, pa1, pi1))

    @pl.when(sid < NS - 1)
    def _():
        pltpu.sync_copy(zwords.at[pl.ds(0, ROWS_T * 3)],
                        acc.at[pl.ds(w0, ROWS_T * 3)])

    @pl.when(sid == NS - 1)
    def _():
        pltpu.sync_copy(zwords.at[pl.ds(0, ROWS_LAST * 3)],
                        acc.at[pl.ds((NS - 1) * ROWS_T * 3, ROWS_LAST * 3)])

    plsc.subcore_barrier()

    iota = lax.iota(_i32, L)
    cols = [jnp.full((L,), j, _i32) for j in range(5)]

    def issue_inputs(blk, st):
        iA_v, iB_v, rA_v, rB_v, pe_v, pa_v, pi_v = st
        b = wid * EW + blk * K
        pltpu.sync_copy(idxA.at[pl.ds(b, K)], iA_v)
        pltpu.sync_copy(idxB.at[pl.ds(b, K)], iB_v)
        pltpu.async_copy(tbl.at[iA_v], rA_v, semI)
        pltpu.async_copy(tbl.at[iB_v], rB_v, semI)
        pltpu.async_copy(pe.at[pl.ds(b, K)], pe_v, semI)
        pltpu.async_copy(pa.at[pl.ds(b, K)], pa_v, semI)
        pltpu.async_copy(pi.at[pl.ds(b, K)], pi_v, semI)

    def drain_inputs(st):
        _, _, rA_v, rB_v, pe_v, pa_v, pi_v = st
        pltpu.make_async_copy(tbl.at[pl.ds(0, K)], rA_v, semI).wait()
        pltpu.make_async_copy(tbl.at[pl.ds(0, K)], rB_v, semI).wait()
        pltpu.make_async_copy(pe.at[pl.ds(0, K)], pe_v, semI).wait()
        pltpu.make_async_copy(pa.at[pl.ds(0, K)], pa_v, semI).wait()
        pltpu.make_async_copy(pi.at[pl.ds(0, K)], pi_v, semI).wait()

    def compute_block(blk, st):
        iA_v, iB_v, rA_v, rB_v, pe_v, pa_v, pi_v = st

        @pl.loop(0, G)
        def _grp(g):
            off = g * L
            rid = off + iota
            rid3 = rid * 3
            uxA = plsc.load_gather(rA_v, [rid, cols[0]])
            uzA = plsc.load_gather(rA_v, [rid, cols[1]])
            thA = plsc.load_gather(rA_v, [rid, cols[2]])
            xA = plsc.load_gather(rA_v, [rid, cols[3]])
            zA = plsc.load_gather(rA_v, [rid, cols[4]])
            uxB = plsc.load_gather(rB_v, [rid, cols[0]])
            uzB = plsc.load_gather(rB_v, [rid, cols[1]])
            thB = plsc.load_gather(rB_v, [rid, cols[2]])
            xB = plsc.load_gather(rB_v, [rid, cols[3]])
            zB = plsc.load_gather(rB_v, [rid, cols[4]])
            pE = pe_v[pl.ds(off, L)]
            pA = pa_v[pl.ds(off, L)]
            pI = pi_v[pl.ds(off, L)]
            na = iA_v[pl.ds(off, L)]
            nb = iB_v[pl.ds(off, L)]

            dx = xB - xA
            dz = zB - zA
            d2 = dx * dx + dz * dz
            r = _rsqrt(d2)
            cc = dx * r
            ss = dz * r
            l0 = d2 * r
            r2 = r * r
            EA = pE * pA
            EI = pE * pI
            EAr = EA * r
            EIr = EI * r
            EIr2 = EI * r2
            EIr3 = EIr * r2

            ua = cc * uxA + ss * uzA
            wa = cc * uzA - ss * uxA
            ub = cc * uxB + ss * uzB
            wb = cc * uzB - ss * uxB
            du = ua - ub
            dw = wa - wb
            tsum = thA + thB

            f0 = EAr * du
            f1 = 12.0 * (EIr3 * dw) + 6.0 * (EIr2 * tsum)
            cdw = 6.0 * (EIr2 * dw)
            f2 = cdw + EIr * (4.0 * thA + 2.0 * thB)
            f5 = cdw + EIr * (2.0 * thA + 4.0 * thB)
            FxA = cc * f0 - ss * f1
            FzA = ss * f0 + cc * f1

            a3 = na * 3
            b3 = nb * 3
            s_ = pl.ds(off, L)
            f0_v[s_] = f0
            f1_v[s_] = f1
            f2_v[s_] = f2
            f5_v[s_] = f5
            ua_v[s_] = ua
            wa_v[s_] = wa
            ta_v[s_] = thA
            ub_v[s_] = ub
            wb_v[s_] = wb
            tb_v[s_] = thB
            fxA_v[s_] = FxA
            fzA_v[s_] = FzA
            l0_v[s_] = l0
            c_v[s_] = cc
            s_v[s_] = ss
            plsc.store_scatter(fgab_v, [rid3], FxA)
            plsc.store_scatter(fgab_v, [rid3 + 1], FzA)
            plsc.store_scatter(fgab_v, [rid3 + 2], f2)
            plsc.store_scatter(fgab_v, [K3 + rid3], -FxA)
            plsc.store_scatter(fgab_v, [K3 + rid3 + 1], -FzA)
            plsc.store_scatter(fgab_v, [K3 + rid3 + 2], f5)
            plsc.store_scatter(iab3_v, [rid3], a3)
            plsc.store_scatter(iab3_v, [rid3 + 1], a3 + 1)
            plsc.store_scatter(iab3_v, [rid3 + 2], a3 + 2)
            plsc.store_scatter(iab3_v, [K3 + rid3], b3)
            plsc.store_scatter(iab3_v, [K3 + rid3 + 1], b3 + 1)
            plsc.store_scatter(iab3_v, [K3 + rid3 + 2], b3 + 2)

        base = wid * EW + blk * K
        sl = pl.ds(base, K)
        outs = [
            pltpu.async_copy(f0_v, o_f0.at[sl], semO),
            pltpu.async_copy(f1_v, o_f1.at[sl], semO),
            pltpu.async_copy(f2_v, o_f2.at[sl], semO),
            pltpu.async_copy(f5_v, o_f5.at[sl], semO),
            pltpu.async_copy(ua_v, o_ua.at[sl], semO),
            pltpu.async_copy(wa_v, o_wa.at[sl], semO),
            pltpu.async_copy(ta_v, o_ta.at[sl], semO),
            pltpu.async_copy(ub_v, o_ub.at[sl], semO),
            pltpu.async_copy(wb_v, o_wb.at[sl], semO),
            pltpu.async_copy(tb_v, o_tb.at[sl], semO),
            pltpu.async_copy(fxA_v, o_fxA.at[sl], semO),
            pltpu.async_copy(fzA_v, o_fzA.at[sl], semO),
            pltpu.async_copy(l0_v, o_l0.at[sl], semO),
            pltpu.async_copy(c_v, o_c.at[sl], semO),
            pltpu.async_copy(s_v, o_s.at[sl], semO),
        ]
        pltpu.sync_copy(fgab_v, acc.at[iab3_v], add=True)
        for d in outs:
            d.wait()

    issue_inputs(jnp.int32(0), sets[0])

    @pl.loop(0, NB // 2)
    def _pair(j):
        b0 = 2 * j
        drain_inputs(sets[0])
        issue_inputs(b0 + 1, sets[1])
        compute_block(b0, sets[0])
        drain_inputs(sets[1])
        issue_inputs(b0 + 2, sets[0])
        compute_block(b0 + 1, sets[1])

    drain_inputs(sets[0])
    compute_block(jnp.int32(NB - 1), sets[0])

    plsc.subcore_barrier()

    @pl.when(sid < NS - 1)
    def _():
        pltpu.sync_copy(acc.at[pl.ds(w0, ROWS_T * 3)],
                        o_part.at[cid, pl.ds(w0, ROWS_T * 3)])

    @pl.when(sid == NS - 1)
    def _():
        pltpu.sync_copy(
            acc.at[pl.ds((NS - 1) * ROWS_T * 3, ROWS_LAST * 3)],
            o_part.at[cid, pl.ds((NS - 1) * ROWS_T * 3, ROWS_LAST * 3)])


def kernel(pred_disp, coords, prop_E, prop_A, prop_I22, connectivity):
    tbl = jnp.concatenate(
        [pred_disp, coords[:, 0:1], coords[:, 2:3],
         jnp.zeros((N, 3), _f32)], axis=1)
    idxA = connectivity[:, 0].astype(_i32)
    idxB = connectivity[:, 1].astype(_i32)
    zwords = jnp.zeros((ROWS_T * 3,), _f32)
    (part, f0, f1, f2, f5, ua, wa, ta, ub, wb, tb,
     fxA, fzA, l0, c, s) = _beam_sc(
        tbl, idxA, idxB, prop_E, prop_A, prop_I22, zwords)
    nodal = (part[0] + part[1]).reshape(N, 3)
    fga = jnp.stack([fxA, fzA, f2], axis=1)
    fgb = jnp.stack([-fxA, -fzA, f5], axis=1)
    fl = jnp.stack([f0, f1, f2, -f0, -f1, f5], axis=1)
    dl = jnp.stack([ua, wa, ta, ub, wb, tb], axis=1)
    ne = -f0
    ve = -f1
    mm = (f5 - f2) * 0.5
    return (nodal, fga, fgb, fl, dl, ne, mm, ve, f2, f5, l0, c, s)
